# baseline (speedup 1.0000x reference)
import functools

import jax
import jax.numpy as jnp
from jax import lax
from jax.experimental import pallas as pl
from jax.experimental.pallas import tpu as pltpu
from jax.experimental.pallas import tpu_sc as plsc

_NODE_NF = 1000000
_HIDDEN = 32
_BATCH = 16384
_FIELDS = 26
_TOTAL = _BATCH * _FIELDS

_NC = 2
_NS = 16
_NW = _NC * _NS
_PW = _TOTAL // _NW
_CHUNK = 128
_NCH = _PW // _CHUNK

_TJ = 800
_TK = 1250
_TBJ = 8
_TOUT = _TBJ * _TK

_mesh = plsc.VectorSubcoreMesh(core_axis_name="c", subcore_axis_name="s")


def _tr_body(x_ref, o_ref):
    for j in range(_TBJ):
        o_ref[pl.ds(j * _TK, _TK), :] = x_ref[:, j, :].T


_tc_transpose = pl.pallas_call(
    _tr_body,
    grid=(_TJ // _TBJ,),
    in_specs=[pl.BlockSpec((_HIDDEN, _TBJ, _TK), lambda i: (0, i, 0))],
    out_specs=pl.BlockSpec((_TOUT, _HIDDEN), lambda i: (i, 0)),
    out_shape=jax.ShapeDtypeStruct((_NODE_NF, _HIDDEN), jnp.float32),
)


@functools.partial(
    pl.kernel,
    mesh=_mesh,
    compiler_params=pltpu.CompilerParams(use_tc_tiling_on_sc=False),
    out_type=jax.ShapeDtypeStruct((_TOTAL, _HIDDEN), jnp.float32),
    scratch_types=[
        pltpu.VMEM((_PW,), jnp.int32),
        pltpu.VMEM((_CHUNK, _HIDDEN), jnp.float32),
        pltpu.VMEM((_CHUNK, _HIDDEN), jnp.float32),
        pltpu.SemaphoreType.DMA,
        pltpu.SemaphoreType.DMA,
        pltpu.SemaphoreType.DMA,
        pltpu.SemaphoreType.DMA,
    ],
)
def _sc_gather_rows(zf_hbm, trm_hbm, out_hbm, idx, bufa, bufb,
                    gsa, gsb, osa, osb):
    wid = lax.axis_index("s") * _NC + lax.axis_index("c")
    base = wid * _PW

    pltpu.sync_copy(zf_hbm.at[pl.ds(base, _PW)], idx)

    def gather_copy(c, buf, sem):
        return pltpu.make_async_copy(
            trm_hbm.at[idx.at[pl.ds(c * _CHUNK, _CHUNK)]], buf, sem)

    def write_copy(c, buf, sem):
        return pltpu.make_async_copy(
            buf, out_hbm.at[pl.ds(base + c * _CHUNK, _CHUNK)], sem)

    gather_copy(0, bufa, gsa).start()
    gather_copy(1, bufb, gsb).start()

    def body(i, carry):
        c = 2 * i
        gather_copy(c, bufa, gsa).wait()
        write_copy(c, bufa, osa).start()
        gather_copy(c + 1, bufb, gsb).wait()
        write_copy(c + 1, bufb, osb).start()
        write_copy(c, bufa, osa).wait()
        gather_copy(c + 2, bufa, gsa).start()
        write_copy(c + 1, bufb, osb).wait()
        gather_copy(c + 3, bufb, gsb).start()
        return carry

    lax.fori_loop(0, _NCH // 2 - 1, body, 0)

    last = _NCH - 2
    gather_copy(last, bufa, gsa).wait()
    write_copy(last, bufa, osa).start()
    gather_copy(last + 1, bufb, gsb).wait()
    write_copy(last + 1, bufb, osb).start()
    write_copy(last, bufa, osa).wait()
    write_copy(last + 1, bufb, osb).wait()


def kernel(z, table):
    tt3 = table.T.reshape(_HIDDEN, _TJ, _TK)
    trm = _tc_transpose(tt3)
    zf = z.T.reshape(_TOTAL)
    out = _sc_gather_rows(zf, trm)
    out = out.reshape(_FIELDS, _BATCH, _HIDDEN).transpose(1, 0, 2)
    return (out, 0)


# speedup vs baseline: 1.3731x; 1.3731x over previous
import functools

import jax
import jax.numpy as jnp
from jax import lax
from jax.experimental import pallas as pl
from jax.experimental.pallas import tpu as pltpu
from jax.experimental.pallas import tpu_sc as plsc

_NODE_NF = 1000000
_HIDDEN = 32
_BATCH = 16384
_FIELDS = 26
_TOTAL = _BATCH * _FIELDS

_NC = 2
_NS = 16
_NW = _NC * _NS
_PW = _TOTAL // _NW
_CHUNK = 128
_NCH = _PW // _CHUNK

_mesh = plsc.VectorSubcoreMesh(core_axis_name="c", subcore_axis_name="s")


@functools.partial(
    pl.kernel,
    mesh=_mesh,
    compiler_params=pltpu.CompilerParams(use_tc_tiling_on_sc=False),
    out_type=jax.ShapeDtypeStruct((_TOTAL, _HIDDEN), jnp.float32),
    scratch_types=[
        pltpu.VMEM((_PW,), jnp.int32),
        pltpu.VMEM((_CHUNK, _HIDDEN), jnp.float32),
        pltpu.VMEM((_CHUNK, _HIDDEN), jnp.float32),
        pltpu.SemaphoreType.DMA,
        pltpu.SemaphoreType.DMA,
        pltpu.SemaphoreType.DMA,
        pltpu.SemaphoreType.DMA,
    ],
)
def _sc_gather_rows(zf_hbm, table_hbm, out_hbm, idx, bufa, bufb,
                    gsa, gsb, osa, osb):
    wid = lax.axis_index("s") * _NC + lax.axis_index("c")
    base = wid * _PW

    pltpu.sync_copy(zf_hbm.at[pl.ds(base, _PW)], idx)

    def fire_gather(c, buf, sem):
        pltpu.async_copy(
            table_hbm.at[idx.at[pl.ds(c * _CHUNK, _CHUNK)]], buf, sem)

    def drain_gather(c, buf, sem):
        pltpu.make_async_copy(
            table_hbm.at[idx.at[pl.ds(c * _CHUNK, _CHUNK)]], buf, sem).wait()

    def fire_write(c, buf, sem):
        pltpu.async_copy(
            buf, out_hbm.at[pl.ds(base + c * _CHUNK, _CHUNK)], sem)

    def drain_write(c, buf, sem):
        pltpu.make_async_copy(
            buf, out_hbm.at[pl.ds(base + c * _CHUNK, _CHUNK)], sem).wait()

    fire_gather(0, bufa, gsa)
    fire_gather(1, bufb, gsb)

    def body(i, carry):
        c = 2 * i
        drain_gather(c, bufa, gsa)
        fire_write(c, bufa, osa)
        drain_gather(c + 1, bufb, gsb)
        fire_write(c + 1, bufb, osb)
        drain_write(c, bufa, osa)
        fire_gather(c + 2, bufa, gsa)
        drain_write(c + 1, bufb, osb)
        fire_gather(c + 3, bufb, gsb)
        return carry

    lax.fori_loop(0, _NCH // 2 - 1, body, 0)

    last = _NCH - 2
    drain_gather(last, bufa, gsa)
    fire_write(last, bufa, osa)
    drain_gather(last + 1, bufb, gsb)
    fire_write(last + 1, bufb, osb)
    drain_write(last, bufa, osa)
    drain_write(last + 1, bufb, osb)


def kernel(z, table):
    zf = z.reshape(_TOTAL)
    out = _sc_gather_rows(zf, table)
    return (out.reshape(_BATCH, _FIELDS, _HIDDEN), 0)


# speedup vs baseline: 1.4201x; 1.0342x over previous
import functools

import jax
import jax.numpy as jnp
from jax import lax
from jax.experimental import pallas as pl
from jax.experimental.pallas import tpu as pltpu
from jax.experimental.pallas import tpu_sc as plsc

_NODE_NF = 1000000
_HIDDEN = 32
_BATCH = 16384
_FIELDS = 26
_TOTAL = _BATCH * _FIELDS

_NC = 2
_NS = 16
_NW = _NC * _NS
_PW = _TOTAL // _NW
_CHUNK = 128
_NCH = _PW // _CHUNK
_GC = 13
_GROW = _GC * _CHUNK
_NG = _NCH // _GC

_mesh = plsc.VectorSubcoreMesh(core_axis_name="c", subcore_axis_name="s")


@functools.partial(
    pl.kernel,
    mesh=_mesh,
    compiler_params=pltpu.CompilerParams(use_tc_tiling_on_sc=False),
    out_type=jax.ShapeDtypeStruct((_TOTAL, _HIDDEN), jnp.float32),
    scratch_types=[
        pltpu.VMEM((_PW,), jnp.int32),
        pltpu.VMEM((_GROW, _HIDDEN), jnp.float32),
        pltpu.VMEM((_GROW, _HIDDEN), jnp.float32),
        pltpu.SemaphoreType.DMA,
        pltpu.SemaphoreType.DMA,
        pltpu.SemaphoreType.DMA,
        pltpu.SemaphoreType.DMA,
    ],
)
def _sc_gather_rows(zf_hbm, table_hbm, out_hbm, idx, bufa, bufb,
                    gsa, gsb, osa, osb):
    wid = lax.axis_index("s") * _NC + lax.axis_index("c")
    base = wid * _PW

    pltpu.sync_copy(zf_hbm.at[pl.ds(base, _PW)], idx)

    def fire_gather(g, buf, sem):
        for k in range(_GC):
            pltpu.async_copy(
                table_hbm.at[idx.at[pl.ds((g * _GC + k) * _CHUNK, _CHUNK)]],
                buf.at[pl.ds(k * _CHUNK, _CHUNK)],
                sem)

    def drain_gather(g, buf, sem):
        for k in range(_GC):
            pltpu.make_async_copy(
                table_hbm.at[idx.at[pl.ds((g * _GC + k) * _CHUNK, _CHUNK)]],
                buf.at[pl.ds(k * _CHUNK, _CHUNK)],
                sem).wait()

    def fire_write(g, buf, sem):
        pltpu.async_copy(
            buf, out_hbm.at[pl.ds(base + g * _GROW, _GROW)], sem)

    def drain_write(g, buf, sem):
        pltpu.make_async_copy(
            buf, out_hbm.at[pl.ds(base + g * _GROW, _GROW)], sem).wait()

    fire_gather(0, bufa, gsa)
    fire_gather(1, bufb, gsb)

    def body(i, carry):
        c = 2 * i
        drain_gather(c, bufa, gsa)
        fire_write(c, bufa, osa)
        drain_gather(c + 1, bufb, gsb)
        fire_write(c + 1, bufb, osb)
        drain_write(c, bufa, osa)
        fire_gather(c + 2, bufa, gsa)
        drain_write(c + 1, bufb, osb)
        fire_gather(c + 3, bufb, gsb)
        return carry

    lax.fori_loop(0, _NG // 2 - 1, body, 0)

    last = _NG - 2
    drain_gather(last, bufa, gsa)
    fire_write(last, bufa, osa)
    drain_gather(last + 1, bufb, gsb)
    fire_write(last + 1, bufb, osb)
    drain_write(last, bufa, osa)
    drain_write(last + 1, bufb, osb)


def kernel(z, table):
    zf = z.reshape(_TOTAL)
    out = _sc_gather_rows(zf, table)
    return (out.reshape(_BATCH, _FIELDS, _HIDDEN), 0)
